# baseline (speedup 1.0000x reference)
import jax
import jax.numpy as jnp
from jax.experimental import pallas as pl
from jax.experimental.pallas import tpu as pltpu

LATENT_DIM = 1024
CODEBOOK_SIZE = 1024
BETA = 0.25
BLOCK_ROWS = 1024
TOTAL_ROWS = 8192
NUM_BLOCKS = TOTAL_ROWS // BLOCK_ROWS


def _vq_kernel(x_ref, w_ref, wt_ref, qt_ref, enc_ref, idx_ref, loss_ref,
               perp_ref, acc_ref, cnt_ref):
    i = pl.program_id(0)
    xb = x_ref[...]
    w = w_ref[...]
    wt = wt_ref[...]

    xsq = jnp.sum(xb * xb, axis=-1, keepdims=True)
    colsq = jnp.sum(w * w, axis=0, keepdims=True)
    s = jax.lax.dot_general(xb, w, (((1,), (1,)), ((), ())))
    distances = (xsq + colsq) - 2.0 * s

    dmin = jnp.min(distances, axis=1, keepdims=True)
    code_iota = jax.lax.broadcasted_iota(jnp.int32, distances.shape, 1)
    idx2d = jnp.min(jnp.where(distances == dmin, code_iota, CODEBOOK_SIZE),
                    axis=1, keepdims=True)

    enc = (code_iota == idx2d).astype(jnp.float32)
    enc_ref[...] = enc
    idx_ref[...] = idx2d

    qt_ref[...] = jax.lax.dot_general(wt, enc, (((1,), (1,)), ((), ())))

    rowsq = jnp.sum(w * w, axis=1, keepdims=True)
    rowsq_at = jax.lax.dot_general(enc, rowsq, (((1,), (0,)), ((), ())))
    s_at = jnp.sum(enc * s, axis=1, keepdims=True)

    @pl.when(i == 0)
    def _init():
        acc_ref[...] = jnp.zeros_like(acc_ref)
        cnt_ref[...] = jnp.zeros_like(cnt_ref)

    acc_ref[...] += jnp.sum(rowsq_at - 2.0 * s_at + xsq, keepdims=True)
    cnt_ref[...] += jnp.sum(enc, axis=0, keepdims=True)

    @pl.when(i == NUM_BLOCKS - 1)
    def _finalize():
        m = acc_ref[...] / jnp.float32(TOTAL_ROWS * LATENT_DIM)
        loss_ref[...] = m + jnp.float32(BETA) * m
        avg = cnt_ref[...] / jnp.float32(TOTAL_ROWS)
        ent = jnp.sum(avg * jnp.log(avg + 1e-10), keepdims=True)
        perp_ref[...] = jnp.exp(-ent)


@jax.jit
def kernel(x, W):
    xp = jnp.transpose(x, (0, 2, 3, 1))
    xf = xp.reshape(TOTAL_ROWS, LATENT_DIM)

    out_shapes = (
        jax.ShapeDtypeStruct((TOTAL_ROWS, BLOCK_ROWS), jnp.float32),
        jax.ShapeDtypeStruct((TOTAL_ROWS, CODEBOOK_SIZE), jnp.float32),
        jax.ShapeDtypeStruct((TOTAL_ROWS, 1), jnp.int32),
        jax.ShapeDtypeStruct((1, 1), jnp.float32),
        jax.ShapeDtypeStruct((1, 1), jnp.float32),
    )
    qt, enc, idx, loss, perp = pl.pallas_call(
        _vq_kernel,
        grid=(NUM_BLOCKS,),
        in_specs=[
            pl.BlockSpec((BLOCK_ROWS, LATENT_DIM), lambda i: (i, 0)),
            pl.BlockSpec((CODEBOOK_SIZE, LATENT_DIM), lambda i: (0, 0)),
            pl.BlockSpec((LATENT_DIM, CODEBOOK_SIZE), lambda i: (0, 0)),
        ],
        out_specs=(
            pl.BlockSpec((LATENT_DIM, BLOCK_ROWS), lambda i: (i, 0)),
            pl.BlockSpec((BLOCK_ROWS, CODEBOOK_SIZE), lambda i: (i, 0)),
            pl.BlockSpec((BLOCK_ROWS, 1), lambda i: (i, 0)),
            pl.BlockSpec((1, 1), lambda i: (0, 0)),
            pl.BlockSpec((1, 1), lambda i: (0, 0)),
        ),
        scratch_shapes=[
            pltpu.VMEM((1, 1), jnp.float32),
            pltpu.VMEM((1, CODEBOOK_SIZE), jnp.float32),
        ],
        out_shape=out_shapes,
    )(xf, W, W.T)

    q_out = qt.reshape(x.shape)
    return (loss[0, 0], q_out, perp[0, 0], enc, idx)


# speedup vs baseline: 2.8121x; 2.8121x over previous
import jax
import jax.numpy as jnp
from jax.experimental import pallas as pl
from jax.experimental.pallas import tpu as pltpu

LATENT_DIM = 1024
CODEBOOK_SIZE = 1024
BETA = 0.25
BLOCK_ROWS = 1024
TOTAL_ROWS = 8192
NUM_BLOCKS = TOTAL_ROWS // BLOCK_ROWS


def _vq_kernel(x_ref, w_ref, qst_ref, enc_ref, idx_ref, loss_ref, perp_ref,
               acc_ref, cnt_ref, colsq_ref):
    i = pl.program_id(0)
    xb = x_ref[...]
    w = w_ref[...]

    @pl.when(i == 0)
    def _init():
        colsq_ref[...] = jnp.sum(w * w, axis=0, keepdims=True)
        acc_ref[...] = jnp.zeros_like(acc_ref)
        cnt_ref[...] = jnp.zeros_like(cnt_ref)

    xsq = jnp.sum(xb * xb, axis=-1, keepdims=True)
    s = jax.lax.dot_general(xb, w, (((1,), (1,)), ((), ())))
    distances = (xsq + colsq_ref[...]) - 2.0 * s

    dmin = jnp.min(distances, axis=1, keepdims=True)
    code_iota = jax.lax.broadcasted_iota(jnp.int32, distances.shape, 1)
    idx2d = jnp.min(jnp.where(distances == dmin, code_iota, CODEBOOK_SIZE),
                    axis=1, keepdims=True)

    enc = (code_iota == idx2d).astype(jnp.float32)
    enc_ref[...] = enc
    idx_ref[...] = idx2d

    qst_ref[...] = jax.lax.dot_general(enc, w, (((1,), (0,)), ((), ())))

    acc_ref[...] += jnp.sum(dmin, keepdims=True)
    cnt_ref[...] += jnp.sum(enc, axis=0, keepdims=True)

    @pl.when(i == NUM_BLOCKS - 1)
    def _finalize():
        rowsq = jnp.sum(w * w, axis=1, keepdims=True)
        cnt = cnt_ref[...]
        corr = (jax.lax.dot_general(cnt, rowsq, (((1,), (0,)), ((), ())))
                - jnp.sum(cnt * colsq_ref[...], keepdims=True))
        m = (acc_ref[...] + corr) / jnp.float32(TOTAL_ROWS * LATENT_DIM)
        loss_ref[...] = m + jnp.float32(BETA) * m
        avg = cnt / jnp.float32(TOTAL_ROWS)
        ent = jnp.sum(avg * jnp.log(avg + 1e-10), keepdims=True)
        perp_ref[...] = jnp.exp(-ent)


@jax.jit
def kernel(x, W):
    xp = jnp.transpose(x, (0, 2, 3, 1))
    input_shape = xp.shape
    xf = xp.reshape(TOTAL_ROWS, LATENT_DIM)

    out_shapes = (
        jax.ShapeDtypeStruct((TOTAL_ROWS, LATENT_DIM), jnp.float32),
        jax.ShapeDtypeStruct((TOTAL_ROWS, CODEBOOK_SIZE), jnp.float32),
        jax.ShapeDtypeStruct((TOTAL_ROWS, 1), jnp.int32),
        jax.ShapeDtypeStruct((1, 1), jnp.float32),
        jax.ShapeDtypeStruct((1, 1), jnp.float32),
    )
    qst, enc, idx, loss, perp = pl.pallas_call(
        _vq_kernel,
        grid=(NUM_BLOCKS,),
        in_specs=[
            pl.BlockSpec((BLOCK_ROWS, LATENT_DIM), lambda i: (i, 0)),
            pl.BlockSpec((CODEBOOK_SIZE, LATENT_DIM), lambda i: (0, 0)),
        ],
        out_specs=(
            pl.BlockSpec((BLOCK_ROWS, LATENT_DIM), lambda i: (i, 0)),
            pl.BlockSpec((BLOCK_ROWS, CODEBOOK_SIZE), lambda i: (i, 0)),
            pl.BlockSpec((BLOCK_ROWS, 1), lambda i: (i, 0)),
            pl.BlockSpec((1, 1), lambda i: (0, 0)),
            pl.BlockSpec((1, 1), lambda i: (0, 0)),
        ),
        scratch_shapes=[
            pltpu.VMEM((1, 1), jnp.float32),
            pltpu.VMEM((1, CODEBOOK_SIZE), jnp.float32),
            pltpu.VMEM((1, CODEBOOK_SIZE), jnp.float32),
        ],
        out_shape=out_shapes,
    )(xf, W)

    q_out = jnp.transpose(qst.reshape(input_shape), (0, 3, 1, 2))
    return (loss[0, 0], q_out, perp[0, 0], enc, idx)
